# baseline (speedup 1.0000x reference)
import functools

import jax
import jax.numpy as jnp
from jax import lax
from jax.experimental import pallas as pl
from jax.experimental.pallas import tpu as pltpu

N_CLASSES = 1000000
EMBED_DIM = 64
BATCH = 16384

_RB = 512
_G = BATCH // _RB


_NSEM = 8


def _gather_body(idx_sref, table_ref, out_ref, sems):
    g = pl.program_id(0)
    gbase = g * _RB

    def issue(jj, _):
        j = jj * _NSEM
        for q in range(_NSEM):
            i = idx_sref[gbase + j + q]
            pltpu.make_async_copy(
                table_ref.at[pl.ds(i, 1), :],
                out_ref.at[pl.ds(j + q, 1), :],
                sems.at[q],
            ).start()
        return 0

    lax.fori_loop(0, _RB // _NSEM, issue, 0)
    for q in range(_NSEM):
        pltpu.make_async_copy(
            table_ref.at[pl.ds(0, _RB // _NSEM), :],
            out_ref.at[pl.ds(0, _RB // _NSEM), :],
            sems.at[q],
        ).wait()


@jax.jit
def _embed_gather(labels, table):
    grid_spec = pltpu.PrefetchScalarGridSpec(
        num_scalar_prefetch=1,
        grid=(_G,),
        in_specs=[pl.BlockSpec(memory_space=pl.ANY)],
        out_specs=pl.BlockSpec((_RB, EMBED_DIM), lambda g, idx: (g, 0)),
        scratch_shapes=[pltpu.SemaphoreType.DMA((_NSEM,))],
    )
    return pl.pallas_call(
        _gather_body,
        grid_spec=grid_spec,
        out_shape=jax.ShapeDtypeStruct((BATCH, EMBED_DIM), jnp.float32),
    )(labels, table)


def kernel(class_labels, embedding_table):
    lab = class_labels.astype(jnp.int32)
    out = _embed_gather(lab, embedding_table)
    return out.reshape(BATCH, 1, EMBED_DIM)


# speedup vs baseline: 1.1679x; 1.1679x over previous
import functools

import jax
import jax.numpy as jnp
from jax import lax
from jax.experimental import pallas as pl
from jax.experimental.pallas import tpu as pltpu
from jax.experimental.pallas import tpu_sc as plsc

N_CLASSES = 1000000
EMBED_DIM = 64
BATCH = 16384

_info = plsc.get_sparse_core_info()
_NC, _NS = _info.num_cores, _info.num_subcores
_NW = _NC * _NS
_B_PER_W = BATCH // _NW


@functools.partial(
    pl.kernel,
    mesh=plsc.VectorSubcoreMesh(core_axis_name="c", subcore_axis_name="s"),
    out_type=jax.ShapeDtypeStruct((BATCH, 128), jnp.float32),
    scratch_types=[
        pltpu.VMEM((_B_PER_W,), jnp.int32),
        pltpu.VMEM((_B_PER_W, 128), jnp.float32),
        pltpu.SemaphoreType.DMA,
    ],
)
def _embed_gather(lab_hbm, table_hbm, out_hbm, lab_v, rows_v, sem):
    wid = lax.axis_index("s") * _NC + lax.axis_index("c")
    base = wid * _B_PER_W

    pltpu.sync_copy(lab_hbm.at[wid], lab_v)

    def body(g, _):
        v = lab_v[pl.ds(g * 16, 16)]
        for l in range(16):
            i = v[l]
            j = g * 16 + l
            pltpu.make_async_copy(
                table_hbm.at[i],
                rows_v.at[j, pl.ds(0, EMBED_DIM)],
                sem,
            ).start()
        return 0

    lax.fori_loop(0, _B_PER_W // 16, body, 0)
    pltpu.make_async_copy(
        out_hbm.at[pl.ds(0, _B_PER_W // 2), :],
        rows_v.at[pl.ds(0, _B_PER_W // 2), :],
        sem,
    ).wait()
    pltpu.sync_copy(
        rows_v,
        out_hbm.at[pl.ds(base, _B_PER_W)],
    )


def kernel(class_labels, embedding_table):
    lab = class_labels.astype(jnp.int32).reshape(_NW, _B_PER_W)
    out = _embed_gather(lab, embedding_table)
    return out[:, :EMBED_DIM].reshape(BATCH, 1, EMBED_DIM)
